# baseline (speedup 1.0000x reference)
import jax
import jax.numpy as jnp
from jax.experimental import pallas as pl
from jax.experimental.pallas import tpu as pltpu

_G = 8
_CH = 160000
_PC = 32000


def _make_body(B, L, C, NC):
    TOT = (B // _G) * NC

    def donor_dma(s_r, e_r, dec_r, perm_r, wave_hbm, dbuf, dsem, g, c, slot,
                  do_start):
        for r in range(_G):
            row = g * _G + r
            s = s_r[row]
            e = e_r[row]
            active = (dec_r[row] == 1) & (e > s)
            s128 = (s // 128) * 128
            e128 = ((e + 127) // 128) * 128
            cnt = jnp.where(active, (e128 - s128 + _PC - 1) // _PC, 0)
            p = perm_r[row]

            def piece(j, _, r=r, p=p, s128=s128):
                st = pl.multiple_of(jnp.minimum(s128 + j * _PC, L - _PC), 128)
                cp = pltpu.make_async_copy(
                    wave_hbm.at[p, pl.ds(st, _PC)],
                    dbuf.at[slot, r, pl.ds(st, _PC)],
                    dsem.at[slot, r],
                )
                if do_start:
                    cp.start()
                else:
                    cp.wait()
                return 0

            jax.lax.fori_loop(0, cnt, piece, 0)

    def label_dma(dec_r, perm_r, oh_hbm, ohbuf, ohsem, g, do_start):
        for r in range(_G):
            row = g * _G + r

            @pl.when(dec_r[row] == 1)
            def _():
                cp = pltpu.make_async_copy(
                    oh_hbm.at[perm_r[row]], ohbuf.at[r], ohsem.at[r])
                if do_start:
                    cp.start()
                else:
                    cp.wait()

    def body(s_r, e_r, dec_r, perm_r,
             wave_b, wave_hbm, oh_b, oh_hbm, lam_r,
             out_w, out_oh,
             dbuf, ohbuf, dsem, ohsem):
        g = pl.program_id(0)
        c = pl.program_id(1)
        step = g * NC + c
        slot = jax.lax.rem(step, 2)

        @pl.when(step == 0)
        def _():
            donor_dma(s_r, e_r, dec_r, perm_r, wave_hbm, dbuf, dsem,
                      g, c, slot, True)

        @pl.when(c == 0)
        def _():
            label_dma(dec_r, perm_r, oh_hbm, ohbuf, ohsem, g, True)

        @pl.when(step + 1 < TOT)
        def _():
            wrap = c + 1 == NC
            gn = jnp.where(wrap, g + 1, g)
            cn = jnp.where(wrap, 0, c + 1)
            donor_dma(s_r, e_r, dec_r, perm_r, wave_hbm, dbuf, dsem,
                      gn, cn, 1 - slot, True)

        donor_dma(s_r, e_r, dec_r, perm_r, wave_hbm, dbuf, dsem,
                  g, c, slot, False)

        lo = c * _CH
        svec = jnp.stack([s_r[g * _G + r] for r in range(_G)]).reshape(_G, 1)
        evec = jnp.stack([e_r[g * _G + r] for r in range(_G)]).reshape(_G, 1)
        dvec = jnp.stack([dec_r[g * _G + r] for r in range(_G)]).reshape(_G, 1)
        lenvec = jnp.where(dvec == 1, evec - svec, 0).astype(jnp.uint32)
        any_need = jnp.any((lenvec > 0) & (svec < lo + _CH) & (evec > lo))

        @pl.when(any_need)
        def _():
            pos = jax.lax.broadcasted_iota(jnp.int32, (_G, _CH), 1) + lo
            m = (pos - svec).astype(jnp.uint32) < lenvec
            out_w[...] = jnp.where(m, dbuf[slot], wave_b[...])

        @pl.when(jnp.logical_not(any_need))
        def _():
            out_w[...] = wave_b[...]

        @pl.when(c == NC - 1)
        def _():
            label_dma(dec_r, perm_r, oh_hbm, ohbuf, ohsem, g, False)
            lamv = jnp.stack(
                [lam_r[g * _G + r] for r in range(_G)]).reshape(_G, 1)
            mix = lamv * oh_b[...] + (1.0 - lamv) * ohbuf[...]
            out_oh[...] = jnp.where(dvec == 1, mix, oh_b[...])

    return body


def kernel(wave, onehot_machine, lam, dec, perm, start):
    B, L = wave.shape
    C = onehot_machine.shape[1]
    NC = L // _CH

    crop = ((1.0 - lam) * L).astype(jnp.int32)
    max_start = jnp.maximum(1, L - crop)
    s = jnp.mod(start, max_start)
    e = s + crop
    deci = dec.astype(jnp.int32)

    def wave_map(g, c, *_):
        return g, c

    def oh_map(g, c, *_):
        return g, 0

    grid_spec = pltpu.PrefetchScalarGridSpec(
        num_scalar_prefetch=4,
        grid=(B // _G, NC),
        in_specs=[
            pl.BlockSpec((_G, _CH), wave_map),
            pl.BlockSpec(memory_space=pl.ANY),
            pl.BlockSpec((_G, C), oh_map),
            pl.BlockSpec(memory_space=pl.ANY),
            pl.BlockSpec(memory_space=pltpu.SMEM),
        ],
        out_specs=[
            pl.BlockSpec((_G, _CH), wave_map),
            pl.BlockSpec((_G, C), oh_map),
        ],
        scratch_shapes=[
            pltpu.VMEM((2, _G, _CH), jnp.float32),
            pltpu.VMEM((_G, C), jnp.float32),
            pltpu.SemaphoreType.DMA((2, _G)),
            pltpu.SemaphoreType.DMA((_G,)),
        ],
    )

    out_w, out_oh = pl.pallas_call(
        _make_body(B, L, C, NC),
        grid_spec=grid_spec,
        out_shape=[
            jax.ShapeDtypeStruct((B, L), jnp.float32),
            jax.ShapeDtypeStruct((B, C), jnp.float32),
        ],
    )(s, e, deci, perm, wave, wave, onehot_machine, onehot_machine, lam)
    return out_w, out_oh


# speedup vs baseline: 1.0037x; 1.0037x over previous
import jax
import jax.numpy as jnp
from jax.experimental import pallas as pl
from jax.experimental.pallas import tpu as pltpu

_G = 8
_CH = 160000
_PC = 16000


def _make_body(B, L, C, NC):
    TOT = (B // _G) * NC

    def donor_dma(s_r, e_r, dec_r, perm_r, wave_hbm, dbuf, dsem, g, c, slot,
                  do_start):
        for r in range(_G):
            row = g * _G + r
            s = s_r[row]
            e = e_r[row]
            active = (dec_r[row] == 1) & (e > s)
            s128 = (s // 128) * 128
            e128 = ((e + 127) // 128) * 128
            cnt = jnp.where(active, (e128 - s128 + _PC - 1) // _PC, 0)
            p = perm_r[row]

            def piece(j, _, r=r, p=p, s128=s128):
                st = pl.multiple_of(jnp.minimum(s128 + j * _PC, L - _PC), 128)
                cp = pltpu.make_async_copy(
                    wave_hbm.at[p, pl.ds(st, _PC)],
                    dbuf.at[slot, r, pl.ds(st, _PC)],
                    dsem.at[slot, r],
                )
                if do_start:
                    cp.start()
                else:
                    cp.wait()
                return 0

            jax.lax.fori_loop(0, cnt, piece, 0)

    def label_dma(dec_r, perm_r, oh_hbm, ohbuf, ohsem, g, do_start):
        for r in range(_G):
            row = g * _G + r

            @pl.when(dec_r[row] == 1)
            def _():
                cp = pltpu.make_async_copy(
                    oh_hbm.at[perm_r[row]], ohbuf.at[r], ohsem.at[r])
                if do_start:
                    cp.start()
                else:
                    cp.wait()

    def body(s_r, e_r, dec_r, perm_r,
             wave_b, wave_hbm, oh_b, oh_hbm, lam_r,
             out_w, out_oh,
             dbuf, ohbuf, dsem, ohsem):
        g = pl.program_id(0)
        c = pl.program_id(1)
        step = g * NC + c
        slot = jax.lax.rem(step, 2)

        @pl.when(step == 0)
        def _():
            donor_dma(s_r, e_r, dec_r, perm_r, wave_hbm, dbuf, dsem,
                      g, c, slot, True)

        @pl.when(c == 0)
        def _():
            label_dma(dec_r, perm_r, oh_hbm, ohbuf, ohsem, g, True)

        @pl.when(step + 1 < TOT)
        def _():
            wrap = c + 1 == NC
            gn = jnp.where(wrap, g + 1, g)
            cn = jnp.where(wrap, 0, c + 1)
            donor_dma(s_r, e_r, dec_r, perm_r, wave_hbm, dbuf, dsem,
                      gn, cn, 1 - slot, True)

        donor_dma(s_r, e_r, dec_r, perm_r, wave_hbm, dbuf, dsem,
                  g, c, slot, False)

        lo = c * _CH
        svec = jnp.stack([s_r[g * _G + r] for r in range(_G)]).reshape(_G, 1)
        evec = jnp.stack([e_r[g * _G + r] for r in range(_G)]).reshape(_G, 1)
        dvec = jnp.stack([dec_r[g * _G + r] for r in range(_G)]).reshape(_G, 1)
        lenvec = jnp.where(dvec == 1, evec - svec, 0).astype(jnp.uint32)
        any_need = jnp.any((lenvec > 0) & (svec < lo + _CH) & (evec > lo))

        @pl.when(any_need)
        def _():
            pos = jax.lax.broadcasted_iota(jnp.int32, (_G, _CH), 1) + lo
            m = (pos - svec).astype(jnp.uint32) < lenvec
            out_w[...] = jnp.where(m, dbuf[slot], wave_b[...])

        @pl.when(jnp.logical_not(any_need))
        def _():
            out_w[...] = wave_b[...]

        @pl.when(c == NC - 1)
        def _():
            label_dma(dec_r, perm_r, oh_hbm, ohbuf, ohsem, g, False)
            lamv = jnp.stack(
                [lam_r[g * _G + r] for r in range(_G)]).reshape(_G, 1)
            mix = lamv * oh_b[...] + (1.0 - lamv) * ohbuf[...]
            out_oh[...] = jnp.where(dvec == 1, mix, oh_b[...])

    return body


def kernel(wave, onehot_machine, lam, dec, perm, start):
    B, L = wave.shape
    C = onehot_machine.shape[1]
    NC = L // _CH

    crop = ((1.0 - lam) * L).astype(jnp.int32)
    max_start = jnp.maximum(1, L - crop)
    s = jnp.mod(start, max_start)
    e = s + crop
    deci = dec.astype(jnp.int32)

    def wave_map(g, c, *_):
        return g, c

    def oh_map(g, c, *_):
        return g, 0

    grid_spec = pltpu.PrefetchScalarGridSpec(
        num_scalar_prefetch=4,
        grid=(B // _G, NC),
        in_specs=[
            pl.BlockSpec((_G, _CH), wave_map),
            pl.BlockSpec(memory_space=pl.ANY),
            pl.BlockSpec((_G, C), oh_map),
            pl.BlockSpec(memory_space=pl.ANY),
            pl.BlockSpec(memory_space=pltpu.SMEM),
        ],
        out_specs=[
            pl.BlockSpec((_G, _CH), wave_map),
            pl.BlockSpec((_G, C), oh_map),
        ],
        scratch_shapes=[
            pltpu.VMEM((2, _G, _CH), jnp.float32),
            pltpu.VMEM((_G, C), jnp.float32),
            pltpu.SemaphoreType.DMA((2, _G)),
            pltpu.SemaphoreType.DMA((_G,)),
        ],
    )

    out_w, out_oh = pl.pallas_call(
        _make_body(B, L, C, NC),
        grid_spec=grid_spec,
        out_shape=[
            jax.ShapeDtypeStruct((B, L), jnp.float32),
            jax.ShapeDtypeStruct((B, C), jnp.float32),
        ],
    )(s, e, deci, perm, wave, wave, onehot_machine, onehot_machine, lam)
    return out_w, out_oh
